# Initial kernel scaffold; baseline (speedup 1.0000x reference)
import jax
import jax.numpy as jnp
from jax.experimental import pallas as pl


def kernel(x, edge_index, W1, b1, W2, b2):
    raise NotImplementedError("write your pallas kernel here")


# speedup vs baseline: 11.9206x; 11.9206x over previous
import functools

import jax
import jax.numpy as jnp
from jax import lax
from jax.experimental import pallas as pl
from jax.experimental.pallas import tpu as pltpu
from jax.experimental.pallas import tpu_sc as plsc

NC = 2
NS = 16
NW = NC * NS

CHUNK = 80


def _mesh():
    return plsc.VectorSubcoreMesh(
        core_axis_name="c", subcore_axis_name="s", num_cores=NC, num_subcores=NS
    )


def _deg_kernel(dst, zeros_hist, npad):
    e = dst.shape[0]
    ept = e // NW
    nchunk = ept // CHUNK
    rpt = npad // NS
    half = npad // 2
    nlane = 16

    def body(dst_hbm, zh_hbm, out_hbm, didx, hist, degv, tmp, outv, slots):
        c = lax.axis_index("c")
        s = lax.axis_index("s")
        wid = s * NC + c
        ones16 = jnp.full((nlane,), 1.0, jnp.float32)
        lane_off = lax.iota(jnp.int32, nlane) * half

        for p in range(2):
            lo = p * half
            pltpu.sync_copy(zh_hbm, hist)

            def chunk(i, carry):
                base = wid * ept + i * CHUNK
                pltpu.sync_copy(dst_hbm.at[pl.ds(base, CHUNK)], didx)
                for j in range(CHUNK // nlane):
                    dv = didx[pl.ds(j * nlane, nlane)]
                    m = (dv >= lo) & (dv < lo + half)
                    idx = (dv - lo) + lane_off
                    plsc.addupdate_scatter(hist, [idx], ones16, mask=m)
                return carry

            lax.fori_loop(0, nchunk, chunk, 0)

            def reduce_blk(mb, carry):
                acc = jnp.zeros((nlane,), jnp.float32)
                for l in range(nlane):
                    acc = acc + hist[pl.ds(l * half + mb * nlane, nlane)]
                degv[pl.ds(lo + mb * nlane, nlane)] = acc
                return carry

            lax.fori_loop(0, half // nlane, reduce_blk, 0)

        pltpu.sync_copy(degv, slots.at[pl.ds(s * npad, npad)])
        plsc.subcore_barrier()
        for t in range(NS):
            pltpu.sync_copy(
                slots.at[pl.ds(t * npad + s * rpt, rpt)],
                tmp.at[pl.ds(t * rpt, rpt)],
            )

        def reduce_tiles(mb, carry):
            acc = jnp.zeros((nlane,), jnp.float32)
            for t in range(NS):
                acc = acc + tmp[pl.ds(t * rpt + mb * nlane, nlane)]
            outv[pl.ds(mb * nlane, nlane)] = acc
            return carry

        lax.fori_loop(0, rpt // nlane, reduce_tiles, 0)
        pltpu.sync_copy(outv, out_hbm.at[pl.ds(c * npad + s * rpt, rpt)])

    f = pl.kernel(
        body,
        out_type=jax.ShapeDtypeStruct((NC * npad,), jnp.float32),
        mesh=_mesh(),
        compiler_params=pltpu.CompilerParams(needs_layout_passes=False),
        scratch_types=[
            pltpu.VMEM((CHUNK,), jnp.int32),
            pltpu.VMEM((nlane * half,), jnp.float32),
            pltpu.VMEM((npad,), jnp.float32),
            pltpu.VMEM((NS * rpt,), jnp.float32),
            pltpu.VMEM((rpt,), jnp.float32),
            pltpu.VMEM_SHARED((NS * npad,), jnp.float32),
        ],
    )
    return f(dst, zeros_hist)


def _spmm_kernel(table, src, dst, zeros_feat):
    n, d = table.shape
    e = src.shape[0]
    ept = e // NW
    nchunk = ept // CHUNK
    rpt = n // NS

    def body(tab_hbm, src_hbm, dst_hbm, zeros_hbm, out_hbm, sidx, didx, rows, sem, acc):
        c = lax.axis_index("c")
        s = lax.axis_index("s")
        wid = s * NC + c
        pltpu.sync_copy(
            zeros_hbm.at[pl.ds(s * rpt, rpt)], acc.at[pl.ds(s * rpt, rpt)]
        )
        plsc.subcore_barrier()

        def chunk(i, carry):
            base = wid * ept + i * CHUNK
            pltpu.sync_copy(src_hbm.at[pl.ds(base, CHUNK)], sidx)
            pltpu.sync_copy(dst_hbm.at[pl.ds(base, CHUNK)], didx)
            pltpu.async_copy(tab_hbm.at[sidx], rows, sem).wait()
            pltpu.sync_copy(rows, acc.at[didx], add=True)
            return carry

        lax.fori_loop(0, nchunk, chunk, 0)
        plsc.subcore_barrier()
        pltpu.sync_copy(
            acc.at[pl.ds(s * rpt, rpt)],
            out_hbm.at[pl.ds(c * n + s * rpt, rpt)],
        )

    f = pl.kernel(
        body,
        out_type=jax.ShapeDtypeStruct((NC * n, d), jnp.float32),
        mesh=_mesh(),
        scratch_types=[
            pltpu.VMEM((CHUNK,), jnp.int32),
            pltpu.VMEM((CHUNK,), jnp.int32),
            pltpu.VMEM((CHUNK, d), jnp.float32),
            pltpu.SemaphoreType.DMA,
            pltpu.VMEM_SHARED((n, d), jnp.float32),
        ],
    )
    return f(table, src, dst, zeros_feat)


_BR = 1000


def _scale_body(d0_ref, d1_ref, x_ref, dinv_ref, xs_ref):
    deg = d0_ref[...] + d1_ref[...] + 1.0
    dv = lax.rsqrt(jnp.maximum(deg, 1e-12))
    dinv_ref[...] = dv
    xs_ref[...] = x_ref[...] * dv


def _scale_call(d0, d1, x):
    n, d = x.shape
    grid = n // _BR
    return pl.pallas_call(
        _scale_body,
        grid=(grid,),
        in_specs=[
            pl.BlockSpec((_BR, 1), lambda i: (i, 0)),
            pl.BlockSpec((_BR, 1), lambda i: (i, 0)),
            pl.BlockSpec((_BR, d), lambda i: (i, 0)),
        ],
        out_specs=[
            pl.BlockSpec((_BR, 1), lambda i: (i, 0)),
            pl.BlockSpec((_BR, d), lambda i: (i, 0)),
        ],
        out_shape=[
            jax.ShapeDtypeStruct((n, 1), jnp.float32),
            jax.ShapeDtypeStruct((n, d), jnp.float32),
        ],
    )(d0, d1, x)


def _mid_body(s0_ref, s1_ref, xs_ref, dinv_ref, w1_ref, b1_ref, w2_ref, gs_ref):
    agg = (s0_ref[...] + s1_ref[...] + xs_ref[...]) * dinv_ref[...]
    h = agg @ w1_ref[...] + b1_ref[...]
    h = jnp.maximum(h, 0.0)
    nrm = jnp.sqrt(jnp.sum(h * h, axis=1, keepdims=True))
    h = h / jnp.maximum(nrm, 1e-12)
    gs_ref[...] = (h @ w2_ref[...]) * dinv_ref[...]


def _mid_call(s0, s1, xs, dinv, w1, b1, w2):
    n, d = xs.shape
    dh = w1.shape[1]
    do = w2.shape[1]
    grid = n // _BR
    return pl.pallas_call(
        _mid_body,
        grid=(grid,),
        in_specs=[
            pl.BlockSpec((_BR, d), lambda i: (i, 0)),
            pl.BlockSpec((_BR, d), lambda i: (i, 0)),
            pl.BlockSpec((_BR, d), lambda i: (i, 0)),
            pl.BlockSpec((_BR, 1), lambda i: (i, 0)),
            pl.BlockSpec((d, dh), lambda i: (0, 0)),
            pl.BlockSpec((1, dh), lambda i: (0, 0)),
            pl.BlockSpec((dh, do), lambda i: (0, 0)),
        ],
        out_specs=pl.BlockSpec((_BR, do), lambda i: (i, 0)),
        out_shape=jax.ShapeDtypeStruct((n, do), jnp.float32),
    )(s0, s1, xs, dinv, w1, b1, w2)


def _final_body(t0_ref, t1_ref, gs_ref, dinv_ref, b2_ref, out_ref):
    out_ref[...] = (t0_ref[...] + t1_ref[...] + gs_ref[...]) * dinv_ref[...] + b2_ref[...]


def _final_call(t0, t1, gs, dinv, b2):
    n, d = gs.shape
    grid = n // _BR
    return pl.pallas_call(
        _final_body,
        grid=(grid,),
        in_specs=[
            pl.BlockSpec((_BR, d), lambda i: (i, 0)),
            pl.BlockSpec((_BR, d), lambda i: (i, 0)),
            pl.BlockSpec((_BR, d), lambda i: (i, 0)),
            pl.BlockSpec((_BR, 1), lambda i: (i, 0)),
            pl.BlockSpec((1, d), lambda i: (0, 0)),
        ],
        out_specs=pl.BlockSpec((_BR, d), lambda i: (i, 0)),
        out_shape=jax.ShapeDtypeStruct((n, d), jnp.float32),
    )(t0, t1, gs, dinv, b2)


def kernel(x, edge_index, W1, b1, W2, b2):
    n, d_in = x.shape
    e = edge_index.shape[1]
    assert e % (NW * CHUNK) == 0 and n % NS == 0

    src = edge_index[0]
    dst = edge_index[1]

    npad = ((n + NS * 16 - 1) // (NS * 16)) * (NS * 16)
    zeros_hist = jnp.zeros((16 * (npad // 2),), jnp.float32)
    zeros_feat = jnp.zeros((npad, d_in), jnp.float32)

    degp = _deg_kernel(dst, zeros_hist, npad)
    degp2 = degp.reshape(NC * npad, 1)
    d0 = lax.slice(degp2, (0, 0), (n, 1))
    d1 = lax.slice(degp2, (npad, 0), (npad + n, 1))

    dinv, xs = _scale_call(d0, d1, x)

    xs_p = jnp.pad(xs, ((0, npad - n), (0, 0)))
    sp = _spmm_kernel(xs_p, src, dst, zeros_feat)
    s0 = lax.slice(sp, (0, 0), (n, d_in))
    s1 = lax.slice(sp, (npad, 0), (npad + n, d_in))

    gs = _mid_call(s0, s1, xs, dinv, W1, b1.reshape(1, -1), W2)

    gs_p = jnp.pad(gs, ((0, npad - n), (0, 0)))
    tp = _spmm_kernel(gs_p, src, dst, zeros_feat)
    t0 = lax.slice(tp, (0, 0), (n, gs.shape[1]))
    t1 = lax.slice(tp, (npad, 0), (npad + n, gs.shape[1]))

    return _final_call(t0, t1, gs, dinv, b2.reshape(1, -1))


# speedup vs baseline: 24.9094x; 2.0896x over previous
import functools

import jax
import jax.numpy as jnp
from jax import lax
from jax.experimental import pallas as pl
from jax.experimental.pallas import tpu as pltpu
from jax.experimental.pallas import tpu_sc as plsc

NC = 2
NS = 16
NW = NC * NS

CHUNK = 80


def _mesh():
    return plsc.VectorSubcoreMesh(
        core_axis_name="c", subcore_axis_name="s", num_cores=NC, num_subcores=NS
    )


def _deg_kernel(dst, zeros_hist, npad):
    e = dst.shape[0]
    ept = e // NW
    nchunk = ept // CHUNK
    rpt = npad // NS
    half = npad // 2
    nlane = 16

    def body(dst_hbm, zh_hbm, out_hbm, dstv, hist, degv, tmp, outv, slots):
        c = lax.axis_index("c")
        s = lax.axis_index("s")
        wid = s * NC + c
        ones16 = jnp.full((nlane,), 1.0, jnp.float32)
        lane_off = lax.iota(jnp.int32, nlane) * half

        pltpu.sync_copy(dst_hbm.at[pl.ds(wid * ept, ept)], dstv)
        for p in range(2):
            lo = p * half
            pltpu.sync_copy(zh_hbm, hist)

            def chunk(i, carry):
                dv = dstv[pl.ds(i * nlane, nlane)]
                m = (dv >= lo) & (dv < lo + half)
                idx = (dv - lo) + lane_off
                plsc.addupdate_scatter(hist, [idx], ones16, mask=m)
                return carry

            lax.fori_loop(0, ept // nlane, chunk, 0)

            def reduce_blk(mb, carry):
                acc = jnp.zeros((nlane,), jnp.float32)
                for l in range(nlane):
                    acc = acc + hist[pl.ds(l * half + mb * nlane, nlane)]
                degv[pl.ds(lo + mb * nlane, nlane)] = acc
                return carry

            lax.fori_loop(0, half // nlane, reduce_blk, 0)

        pltpu.sync_copy(degv, slots.at[pl.ds(s * npad, npad)])
        plsc.subcore_barrier()
        for t in range(NS):
            pltpu.sync_copy(
                slots.at[pl.ds(t * npad + s * rpt, rpt)],
                tmp.at[pl.ds(t * rpt, rpt)],
            )

        def reduce_tiles(mb, carry):
            acc = jnp.zeros((nlane,), jnp.float32)
            for t in range(NS):
                acc = acc + tmp[pl.ds(t * rpt + mb * nlane, nlane)]
            outv[pl.ds(mb * nlane, nlane)] = acc
            return carry

        lax.fori_loop(0, rpt // nlane, reduce_tiles, 0)
        pltpu.sync_copy(outv, out_hbm.at[pl.ds(c * npad + s * rpt, rpt)])

    f = pl.kernel(
        body,
        out_type=jax.ShapeDtypeStruct((NC * npad,), jnp.float32),
        mesh=_mesh(),
        compiler_params=pltpu.CompilerParams(needs_layout_passes=False),
        scratch_types=[
            pltpu.VMEM((ept,), jnp.int32),
            pltpu.VMEM((nlane * half,), jnp.float32),
            pltpu.VMEM((npad,), jnp.float32),
            pltpu.VMEM((NS * rpt,), jnp.float32),
            pltpu.VMEM((rpt,), jnp.float32),
            pltpu.VMEM_SHARED((NS * npad,), jnp.float32),
        ],
    )
    return f(dst, zeros_hist)


CH = 125


def _spmm_kernel(table, src2, dst2, zeros_feat):
    n, d = table.shape
    nrows = src2.shape[0]
    cpt = nrows // NW
    rpt = n // NS

    def body(tab_hbm, src_hbm, dst_hbm, zeros_hbm, out_hbm,
             sidx, didx, rows0, rows1, sem0, sem1, acc):
        c = lax.axis_index("c")
        s = lax.axis_index("s")
        wid = s * NC + c
        pltpu.sync_copy(
            zeros_hbm.at[pl.ds(s * rpt, rpt)], acc.at[pl.ds(s * rpt, rpt)]
        )
        plsc.subcore_barrier()

        rows = (rows0, rows1)
        sems = (sem0, sem1)
        hcpt = cpt // 2

        for h in range(2):
            pltpu.sync_copy(src_hbm.at[pl.ds(wid * cpt + h * hcpt, hcpt)], sidx)
            pltpu.sync_copy(dst_hbm.at[pl.ds(wid * cpt + h * hcpt, hcpt)], didx)
            pltpu.async_copy(tab_hbm.at[sidx.at[0]], rows0, sem0)

            def group(g, carry):
                for u in range(2):
                    j = 2 * g + u
                    b = u
                    nb = 1 - u
                    pltpu.make_async_copy(
                        tab_hbm.at[sidx.at[j]], rows[b], sems[b]
                    ).wait()
                    @pl.when(j + 1 < hcpt)
                    def _():
                        pltpu.async_copy(
                            tab_hbm.at[sidx.at[j + 1]], rows[nb], sems[nb]
                        )
                    pltpu.sync_copy(rows[b], acc.at[didx.at[j]], add=True)
                return carry

            lax.fori_loop(0, hcpt // 2, group, 0)
        plsc.subcore_barrier()
        pltpu.sync_copy(
            acc.at[pl.ds(s * rpt, rpt)],
            out_hbm.at[pl.ds(c * n + s * rpt, rpt)],
        )

    f = pl.kernel(
        body,
        out_type=jax.ShapeDtypeStruct((NC * n, d), jnp.float32),
        mesh=_mesh(),
        scratch_types=[
            pltpu.VMEM((cpt // 2, CH), jnp.int32),
            pltpu.VMEM((cpt // 2, CH), jnp.int32),
            pltpu.VMEM((CH, d), jnp.float32),
            pltpu.VMEM((CH, d), jnp.float32),
            pltpu.SemaphoreType.DMA,
            pltpu.SemaphoreType.DMA,
            pltpu.VMEM_SHARED((n, d), jnp.float32),
        ],
    )
    return f(table, src2, dst2, zeros_feat)


_BR = 1000


def _scale_body(d0_ref, d1_ref, x_ref, dinv_ref, xs_ref):
    deg = d0_ref[...] + d1_ref[...] + 1.0
    dv = lax.rsqrt(jnp.maximum(deg, 1e-12))
    dinv_ref[...] = dv
    xs_ref[...] = x_ref[...] * dv


def _scale_call(d0, d1, x):
    n, d = x.shape
    grid = n // _BR
    return pl.pallas_call(
        _scale_body,
        grid=(grid,),
        in_specs=[
            pl.BlockSpec((_BR, 1), lambda i: (i, 0)),
            pl.BlockSpec((_BR, 1), lambda i: (i, 0)),
            pl.BlockSpec((_BR, d), lambda i: (i, 0)),
        ],
        out_specs=[
            pl.BlockSpec((_BR, 1), lambda i: (i, 0)),
            pl.BlockSpec((_BR, d), lambda i: (i, 0)),
        ],
        out_shape=[
            jax.ShapeDtypeStruct((n, 1), jnp.float32),
            jax.ShapeDtypeStruct((n, d), jnp.float32),
        ],
    )(d0, d1, x)


def _mid_body(s0_ref, s1_ref, xs_ref, dinv_ref, w1_ref, b1_ref, w2_ref, gs_ref):
    agg = (s0_ref[...] + s1_ref[...] + xs_ref[...]) * dinv_ref[...]
    h = agg @ w1_ref[...] + b1_ref[...]
    h = jnp.maximum(h, 0.0)
    nrm = jnp.sqrt(jnp.sum(h * h, axis=1, keepdims=True))
    h = h / jnp.maximum(nrm, 1e-12)
    gs_ref[...] = (h @ w2_ref[...]) * dinv_ref[...]


def _mid_call(s0, s1, xs, dinv, w1, b1, w2):
    n, d = xs.shape
    dh = w1.shape[1]
    do = w2.shape[1]
    grid = n // _BR
    return pl.pallas_call(
        _mid_body,
        grid=(grid,),
        in_specs=[
            pl.BlockSpec((_BR, d), lambda i: (i, 0)),
            pl.BlockSpec((_BR, d), lambda i: (i, 0)),
            pl.BlockSpec((_BR, d), lambda i: (i, 0)),
            pl.BlockSpec((_BR, 1), lambda i: (i, 0)),
            pl.BlockSpec((d, dh), lambda i: (0, 0)),
            pl.BlockSpec((1, dh), lambda i: (0, 0)),
            pl.BlockSpec((dh, do), lambda i: (0, 0)),
        ],
        out_specs=pl.BlockSpec((_BR, do), lambda i: (i, 0)),
        out_shape=jax.ShapeDtypeStruct((n, do), jnp.float32),
    )(s0, s1, xs, dinv, w1, b1, w2)


def _final_body(t0_ref, t1_ref, gs_ref, dinv_ref, b2_ref, out_ref):
    out_ref[...] = (t0_ref[...] + t1_ref[...] + gs_ref[...]) * dinv_ref[...] + b2_ref[...]


def _final_call(t0, t1, gs, dinv, b2):
    n, d = gs.shape
    grid = n // _BR
    return pl.pallas_call(
        _final_body,
        grid=(grid,),
        in_specs=[
            pl.BlockSpec((_BR, d), lambda i: (i, 0)),
            pl.BlockSpec((_BR, d), lambda i: (i, 0)),
            pl.BlockSpec((_BR, d), lambda i: (i, 0)),
            pl.BlockSpec((_BR, 1), lambda i: (i, 0)),
            pl.BlockSpec((1, d), lambda i: (0, 0)),
        ],
        out_specs=pl.BlockSpec((_BR, d), lambda i: (i, 0)),
        out_shape=jax.ShapeDtypeStruct((n, d), jnp.float32),
    )(t0, t1, gs, dinv, b2)


def kernel(x, edge_index, W1, b1, W2, b2):
    n, d_in = x.shape
    e = edge_index.shape[1]
    assert e % (NW * CH) == 0 and (e // CH // NW) % 2 == 0 and n % NS == 0

    src = edge_index[0]
    dst = edge_index[1]
    src2 = src.reshape(e // CH, CH)
    dst2 = dst.reshape(e // CH, CH)

    npad = ((n + NS * 16 - 1) // (NS * 16)) * (NS * 16)
    zeros_hist = jnp.zeros((16 * (npad // 2),), jnp.float32)
    zeros_feat = jnp.zeros((npad, d_in), jnp.float32)

    degp = _deg_kernel(dst, zeros_hist, npad)
    degp2 = degp.reshape(NC * npad, 1)
    d0 = lax.slice(degp2, (0, 0), (n, 1))
    d1 = lax.slice(degp2, (npad, 0), (npad + n, 1))

    dinv, xs = _scale_call(d0, d1, x)

    xs_p = jnp.pad(xs, ((0, npad - n), (0, 0)))
    sp = _spmm_kernel(xs_p, src2, dst2, zeros_feat)
    s0 = lax.slice(sp, (0, 0), (n, d_in))
    s1 = lax.slice(sp, (npad, 0), (npad + n, d_in))

    gs = _mid_call(s0, s1, xs, dinv, W1, b1.reshape(1, -1), W2)

    gs_p = jnp.pad(gs, ((0, npad - n), (0, 0)))
    tp = _spmm_kernel(gs_p, src2, dst2, zeros_feat)
    t0 = lax.slice(tp, (0, 0), (n, gs.shape[1]))
    t1 = lax.slice(tp, (npad, 0), (npad + n, gs.shape[1]))

    return _final_call(t0, t1, gs, dinv, b2.reshape(1, -1))


# speedup vs baseline: 25.4204x; 1.0205x over previous
import functools

import jax
import jax.numpy as jnp
from jax import lax
from jax.experimental import pallas as pl
from jax.experimental.pallas import tpu as pltpu
from jax.experimental.pallas import tpu_sc as plsc

NC = 2
NS = 16
NW = NC * NS

CHUNK = 80


def _mesh():
    return plsc.VectorSubcoreMesh(
        core_axis_name="c", subcore_axis_name="s", num_cores=NC, num_subcores=NS
    )


def _deg_kernel(dst, zeros_hist, npad):
    e = dst.shape[0]
    ept = e // NW
    nchunk = ept // CHUNK
    rpt = npad // NS
    half = npad // 2
    nlane = 16

    def body(dst_hbm, zh_hbm, out_hbm, dstv, hist, degv, tmp, outv, slots):
        c = lax.axis_index("c")
        s = lax.axis_index("s")
        wid = s * NC + c
        ones16 = jnp.full((nlane,), 1.0, jnp.float32)
        lane_off = lax.iota(jnp.int32, nlane) * half

        pltpu.sync_copy(dst_hbm.at[pl.ds(wid * ept, ept)], dstv)
        for p in range(2):
            lo = p * half
            pltpu.sync_copy(zh_hbm, hist)

            def chunk(i, carry):
                dv = dstv[pl.ds(i * nlane, nlane)]
                m = (dv >= lo) & (dv < lo + half)
                idx = (dv - lo) + lane_off
                plsc.addupdate_scatter(hist, [idx], ones16, mask=m)
                return carry

            lax.fori_loop(0, ept // nlane, chunk, 0)

            def reduce_blk(mb, carry):
                acc = jnp.zeros((nlane,), jnp.float32)
                for l in range(nlane):
                    acc = acc + hist[pl.ds(l * half + mb * nlane, nlane)]
                degv[pl.ds(lo + mb * nlane, nlane)] = acc
                return carry

            lax.fori_loop(0, half // nlane, reduce_blk, 0)

        pltpu.sync_copy(degv, slots.at[pl.ds(s * npad, npad)])
        plsc.subcore_barrier()
        for t in range(NS):
            pltpu.sync_copy(
                slots.at[pl.ds(t * npad + s * rpt, rpt)],
                tmp.at[pl.ds(t * rpt, rpt)],
            )

        def reduce_tiles(mb, carry):
            acc = jnp.zeros((nlane,), jnp.float32)
            for t in range(NS):
                acc = acc + tmp[pl.ds(t * rpt + mb * nlane, nlane)]
            outv[pl.ds(mb * nlane, nlane)] = acc
            return carry

        lax.fori_loop(0, rpt // nlane, reduce_tiles, 0)
        pltpu.sync_copy(outv, out_hbm.at[pl.ds(c * npad + s * rpt, rpt)])

    f = pl.kernel(
        body,
        out_type=jax.ShapeDtypeStruct((NC * npad,), jnp.float32),
        mesh=_mesh(),
        compiler_params=pltpu.CompilerParams(needs_layout_passes=False),
        scratch_types=[
            pltpu.VMEM((ept,), jnp.int32),
            pltpu.VMEM((nlane * half,), jnp.float32),
            pltpu.VMEM((npad,), jnp.float32),
            pltpu.VMEM((NS * rpt,), jnp.float32),
            pltpu.VMEM((rpt,), jnp.float32),
            pltpu.VMEM_SHARED((NS * npad,), jnp.float32),
        ],
    )
    return f(dst, zeros_hist)


CH = 125


def _spmm_kernel(table, src2, dst2, zeros_feat):
    n, d = table.shape
    nrows = src2.shape[0]
    cpt = nrows // NW
    rpt = n // NS

    def body(tab_hbm, src_hbm, dst_hbm, zeros_hbm, out_hbm,
             sidx, didx, rows0, rows1, gsem0, gsem1, ssem0, ssem1, acc):
        c = lax.axis_index("c")
        s = lax.axis_index("s")
        wid = s * NC + c
        pltpu.sync_copy(
            zeros_hbm.at[pl.ds(s * rpt, rpt)], acc.at[pl.ds(s * rpt, rpt)]
        )
        plsc.subcore_barrier()

        rows = (rows0, rows1)
        gsems = (gsem0, gsem1)
        ssems = (ssem0, ssem1)
        hcpt = cpt // 2

        def wait_gather(j, b):
            pltpu.make_async_copy(tab_hbm.at[sidx.at[j]], rows[b], gsems[b]).wait()

        def wait_scatter(j, b):
            pltpu.make_async_copy(rows[b], acc.at[didx.at[j]], ssems[b]).wait()

        for h in range(2):
            pltpu.sync_copy(src_hbm.at[pl.ds(wid * cpt + h * hcpt, hcpt)], sidx)
            pltpu.sync_copy(dst_hbm.at[pl.ds(wid * cpt + h * hcpt, hcpt)], didx)
            pltpu.async_copy(tab_hbm.at[sidx.at[0]], rows0, gsem0)
            wait_gather(0, 0)
            pltpu.async_copy(tab_hbm.at[sidx.at[1]], rows1, gsem1)
            pltpu.async_copy(rows0, acc.at[didx.at[0]], ssem0, add=True)

            def group(g, carry):
                for u in range(2):
                    j = 2 * g + 1 + u
                    b = 1 - u
                    nb = u
                    wait_gather(j, b)
                    wait_scatter(j - 1, nb)
                    pltpu.async_copy(tab_hbm.at[sidx.at[j + 1]], rows[nb], gsems[nb])
                    pltpu.async_copy(rows[b], acc.at[didx.at[j]], ssems[b], add=True)
                return carry

            lax.fori_loop(0, (hcpt - 2) // 2, group, 0)
            wait_gather(hcpt - 1, 1)
            wait_scatter(hcpt - 2, 0)
            pltpu.sync_copy(rows1, acc.at[didx.at[hcpt - 1]], add=True)
        plsc.subcore_barrier()
        pltpu.sync_copy(
            acc.at[pl.ds(s * rpt, rpt)],
            out_hbm.at[pl.ds(c * n + s * rpt, rpt)],
        )

    f = pl.kernel(
        body,
        out_type=jax.ShapeDtypeStruct((NC * n, d), jnp.float32),
        mesh=_mesh(),
        scratch_types=[
            pltpu.VMEM((cpt // 2, CH), jnp.int32),
            pltpu.VMEM((cpt // 2, CH), jnp.int32),
            pltpu.VMEM((CH, d), jnp.float32),
            pltpu.VMEM((CH, d), jnp.float32),
            pltpu.SemaphoreType.DMA,
            pltpu.SemaphoreType.DMA,
            pltpu.SemaphoreType.DMA,
            pltpu.SemaphoreType.DMA,
            pltpu.VMEM_SHARED((n, d), jnp.float32),
        ],
    )
    return f(table, src2, dst2, zeros_feat)


_BR = 1000


def _scale_body(d0_ref, d1_ref, x_ref, dinv_ref, xs_ref):
    deg = d0_ref[...] + d1_ref[...] + 1.0
    dv = lax.rsqrt(jnp.maximum(deg, 1e-12))
    dinv_ref[...] = dv
    xs_ref[...] = x_ref[...] * dv


def _scale_call(d0, d1, x, npad):
    n, d = x.shape
    grid = n // _BR
    return pl.pallas_call(
        _scale_body,
        grid=(grid,),
        in_specs=[
            pl.BlockSpec((_BR, 1), lambda i: (i, 0)),
            pl.BlockSpec((_BR, 1), lambda i: (i, 0)),
            pl.BlockSpec((_BR, d), lambda i: (i, 0)),
        ],
        out_specs=[
            pl.BlockSpec((_BR, 1), lambda i: (i, 0)),
            pl.BlockSpec((_BR, d), lambda i: (i, 0)),
        ],
        out_shape=[
            jax.ShapeDtypeStruct((n, 1), jnp.float32),
            jax.ShapeDtypeStruct((npad, d), jnp.float32),
        ],
    )(d0, d1, x)


def _mid_body(s0_ref, s1_ref, xs_ref, dinv_ref, w1_ref, b1_ref, w2_ref, gs_ref):
    agg = (s0_ref[...] + s1_ref[...] + xs_ref[...]) * dinv_ref[...]
    h = agg @ w1_ref[...] + b1_ref[...]
    h = jnp.maximum(h, 0.0)
    nrm = jnp.sqrt(jnp.sum(h * h, axis=1, keepdims=True))
    h = h / jnp.maximum(nrm, 1e-12)
    gs_ref[...] = (h @ w2_ref[...]) * dinv_ref[...]


def _mid_call(s0, s1, xs, dinv, w1, b1, w2, npad):
    n = s0.shape[0]
    d = xs.shape[1]
    dh = w1.shape[1]
    do = w2.shape[1]
    grid = n // _BR
    return pl.pallas_call(
        _mid_body,
        grid=(grid,),
        in_specs=[
            pl.BlockSpec((_BR, d), lambda i: (i, 0)),
            pl.BlockSpec((_BR, d), lambda i: (i, 0)),
            pl.BlockSpec((_BR, d), lambda i: (i, 0)),
            pl.BlockSpec((_BR, 1), lambda i: (i, 0)),
            pl.BlockSpec((d, dh), lambda i: (0, 0)),
            pl.BlockSpec((1, dh), lambda i: (0, 0)),
            pl.BlockSpec((dh, do), lambda i: (0, 0)),
        ],
        out_specs=pl.BlockSpec((_BR, do), lambda i: (i, 0)),
        out_shape=jax.ShapeDtypeStruct((npad, do), jnp.float32),
    )(s0, s1, xs, dinv, w1, b1, w2)


def _final_body(t0_ref, t1_ref, gs_ref, dinv_ref, b2_ref, out_ref):
    out_ref[...] = (t0_ref[...] + t1_ref[...] + gs_ref[...]) * dinv_ref[...] + b2_ref[...]


def _final_call(t0, t1, gs, dinv, b2):
    n, d = t0.shape
    grid = n // _BR
    return pl.pallas_call(
        _final_body,
        grid=(grid,),
        in_specs=[
            pl.BlockSpec((_BR, d), lambda i: (i, 0)),
            pl.BlockSpec((_BR, d), lambda i: (i, 0)),
            pl.BlockSpec((_BR, d), lambda i: (i, 0)),
            pl.BlockSpec((_BR, 1), lambda i: (i, 0)),
            pl.BlockSpec((1, d), lambda i: (0, 0)),
        ],
        out_specs=pl.BlockSpec((_BR, d), lambda i: (i, 0)),
        out_shape=jax.ShapeDtypeStruct((n, d), jnp.float32),
    )(t0, t1, gs, dinv, b2)


def kernel(x, edge_index, W1, b1, W2, b2):
    n, d_in = x.shape
    e = edge_index.shape[1]
    assert e % (NW * CH) == 0 and (e // CH // NW) % 2 == 0 and n % NS == 0

    src = edge_index[0]
    dst = edge_index[1]
    src2 = src.reshape(e // CH, CH)
    dst2 = dst.reshape(e // CH, CH)

    npad = ((n + NS * 16 - 1) // (NS * 16)) * (NS * 16)
    zeros_hist = jnp.zeros((16 * (npad // 2),), jnp.float32)
    zeros_feat = jnp.zeros((npad, d_in), jnp.float32)

    degp = _deg_kernel(dst, zeros_hist, npad)
    degp2 = degp.reshape(NC * npad, 1)
    d0 = lax.slice(degp2, (0, 0), (n, 1))
    d1 = lax.slice(degp2, (npad, 0), (npad + n, 1))

    dinv, xs = _scale_call(d0, d1, x, npad)

    sp = _spmm_kernel(xs, src2, dst2, zeros_feat)
    s0 = lax.slice(sp, (0, 0), (n, d_in))
    s1 = lax.slice(sp, (npad, 0), (npad + n, d_in))

    gs = _mid_call(s0, s1, xs, dinv, W1, b1.reshape(1, -1), W2, npad)

    tp = _spmm_kernel(gs, src2, dst2, zeros_feat)
    d_out = gs.shape[1]
    t0 = lax.slice(tp, (0, 0), (n, d_out))
    t1 = lax.slice(tp, (npad, 0), (npad + n, d_out))

    return _final_call(t0, t1, gs, dinv, b2.reshape(1, -1))


# speedup vs baseline: 27.0896x; 1.0657x over previous
import functools

import jax
import jax.numpy as jnp
from jax import lax
from jax.experimental import pallas as pl
from jax.experimental.pallas import tpu as pltpu
from jax.experimental.pallas import tpu_sc as plsc

NC = 2
NS = 16
NW = NC * NS

CHUNK = 80


def _mesh():
    return plsc.VectorSubcoreMesh(
        core_axis_name="c", subcore_axis_name="s", num_cores=NC, num_subcores=NS
    )


def _deg_kernel(dst, zeros_hist, npad):
    e = dst.shape[0]
    ept = e // NW
    nchunk = ept // CHUNK
    rpt = npad // NS
    half = npad // 2
    nlane = 16

    unroll = 5

    def body(dst_hbm, zh_hbm, out_hbm, dstv, hist, degv, tmp, outv, slots):
        c = lax.axis_index("c")
        s = lax.axis_index("s")
        wid = s * NC + c
        ones16 = jnp.full((nlane,), 1.0, jnp.float32)
        lane_ids = lax.iota(jnp.int32, nlane)
        lane_off = (lane_ids % 8) * npad
        m_lo = lane_ids < 8
        m_hi = lane_ids >= 8

        pltpu.sync_copy(dst_hbm.at[pl.ds(wid * ept, ept)], dstv)
        pltpu.sync_copy(zh_hbm, hist)

        def chunk(i, carry):
            for u in range(unroll):
                dv = dstv[pl.ds((i * unroll + u) * nlane, nlane)]
                idx = dv + lane_off
                plsc.addupdate_scatter(hist, [idx], ones16, mask=m_lo)
                plsc.addupdate_scatter(hist, [idx], ones16, mask=m_hi)
            return carry

        lax.fori_loop(0, ept // (nlane * unroll), chunk, 0)

        def reduce_blk(mb, carry):
            for u in range(2):
                acc = jnp.zeros((nlane,), jnp.float32)
                for l in range(8):
                    acc = acc + hist[pl.ds(l * npad + (2 * mb + u) * nlane, nlane)]
                degv[pl.ds((2 * mb + u) * nlane, nlane)] = acc
            return carry

        lax.fori_loop(0, npad // nlane // 2, reduce_blk, 0)

        pltpu.sync_copy(degv, slots.at[pl.ds(s * npad, npad)])
        plsc.subcore_barrier()
        for t in range(NS):
            pltpu.sync_copy(
                slots.at[pl.ds(t * npad + s * rpt, rpt)],
                tmp.at[pl.ds(t * rpt, rpt)],
            )

        def reduce_tiles(mb, carry):
            acc = jnp.zeros((nlane,), jnp.float32)
            for t in range(NS):
                acc = acc + tmp[pl.ds(t * rpt + mb * nlane, nlane)]
            outv[pl.ds(mb * nlane, nlane)] = acc
            return carry

        lax.fori_loop(0, rpt // nlane, reduce_tiles, 0)
        pltpu.sync_copy(outv, out_hbm.at[pl.ds(c * npad + s * rpt, rpt)])

    f = pl.kernel(
        body,
        out_type=jax.ShapeDtypeStruct((NC * npad,), jnp.float32),
        mesh=_mesh(),
        compiler_params=pltpu.CompilerParams(needs_layout_passes=False),
        scratch_types=[
            pltpu.VMEM((ept,), jnp.int32),
            pltpu.VMEM((nlane * half,), jnp.float32),
            pltpu.VMEM((npad,), jnp.float32),
            pltpu.VMEM((NS * rpt,), jnp.float32),
            pltpu.VMEM((rpt,), jnp.float32),
            pltpu.VMEM_SHARED((NS * npad,), jnp.float32),
        ],
    )
    return f(dst, zeros_hist)


CH = 125


def _spmm_kernel(table, src2, dst2, zeros_feat):
    n, d = table.shape
    nrows = src2.shape[0]
    cpt = nrows // NW
    rpt = n // NS

    def body(tab_hbm, src_hbm, dst_hbm, zeros_hbm, out_hbm,
             sidx, didx, rows0, rows1, gsem0, gsem1, ssem0, ssem1, acc):
        c = lax.axis_index("c")
        s = lax.axis_index("s")
        wid = s * NC + c
        pltpu.sync_copy(
            zeros_hbm.at[pl.ds(s * rpt, rpt)], acc.at[pl.ds(s * rpt, rpt)]
        )
        plsc.subcore_barrier()

        rows = (rows0, rows1)
        gsems = (gsem0, gsem1)
        ssems = (ssem0, ssem1)
        hcpt = cpt // 2

        def wait_gather(j, b):
            pltpu.make_async_copy(tab_hbm.at[sidx.at[j]], rows[b], gsems[b]).wait()

        def wait_scatter(j, b):
            pltpu.make_async_copy(rows[b], acc.at[didx.at[j]], ssems[b]).wait()

        for h in range(2):
            pltpu.sync_copy(src_hbm.at[pl.ds(wid * cpt + h * hcpt, hcpt)], sidx)
            pltpu.sync_copy(dst_hbm.at[pl.ds(wid * cpt + h * hcpt, hcpt)], didx)
            pltpu.async_copy(tab_hbm.at[sidx.at[0]], rows0, gsem0)
            wait_gather(0, 0)
            pltpu.async_copy(tab_hbm.at[sidx.at[1]], rows1, gsem1)
            pltpu.async_copy(rows0, acc.at[didx.at[0]], ssem0, add=True)

            def group(g, carry):
                for u in range(2):
                    j = 2 * g + 1 + u
                    b = 1 - u
                    nb = u
                    wait_gather(j, b)
                    wait_scatter(j - 1, nb)
                    pltpu.async_copy(tab_hbm.at[sidx.at[j + 1]], rows[nb], gsems[nb])
                    pltpu.async_copy(rows[b], acc.at[didx.at[j]], ssems[b], add=True)
                return carry

            lax.fori_loop(0, (hcpt - 2) // 2, group, 0)
            wait_gather(hcpt - 1, 1)
            wait_scatter(hcpt - 2, 0)
            pltpu.sync_copy(rows1, acc.at[didx.at[hcpt - 1]], add=True)
        plsc.subcore_barrier()
        pltpu.sync_copy(
            acc.at[pl.ds(s * rpt, rpt)],
            out_hbm.at[pl.ds(c * n + s * rpt, rpt)],
        )

    f = pl.kernel(
        body,
        out_type=jax.ShapeDtypeStruct((NC * n, d), jnp.float32),
        mesh=_mesh(),
        scratch_types=[
            pltpu.VMEM((cpt // 2, CH), jnp.int32),
            pltpu.VMEM((cpt // 2, CH), jnp.int32),
            pltpu.VMEM((CH, d), jnp.float32),
            pltpu.VMEM((CH, d), jnp.float32),
            pltpu.SemaphoreType.DMA,
            pltpu.SemaphoreType.DMA,
            pltpu.SemaphoreType.DMA,
            pltpu.SemaphoreType.DMA,
            pltpu.VMEM_SHARED((n, d), jnp.float32),
        ],
    )
    return f(table, src2, dst2, zeros_feat)


_BR = 1000


def _scale_body(d0_ref, d1_ref, x_ref, dinv_ref, xs_ref):
    deg = d0_ref[...] + d1_ref[...] + 1.0
    dv = lax.rsqrt(jnp.maximum(deg, 1e-12))
    dinv_ref[...] = dv
    xs_ref[...] = x_ref[...] * dv


def _scale_call(d0, d1, x, npad):
    n, d = x.shape
    grid = n // _BR
    return pl.pallas_call(
        _scale_body,
        grid=(grid,),
        in_specs=[
            pl.BlockSpec((_BR, 1), lambda i: (i, 0)),
            pl.BlockSpec((_BR, 1), lambda i: (i, 0)),
            pl.BlockSpec((_BR, d), lambda i: (i, 0)),
        ],
        out_specs=[
            pl.BlockSpec((_BR, 1), lambda i: (i, 0)),
            pl.BlockSpec((_BR, d), lambda i: (i, 0)),
        ],
        out_shape=[
            jax.ShapeDtypeStruct((n, 1), jnp.float32),
            jax.ShapeDtypeStruct((npad, d), jnp.float32),
        ],
    )(d0, d1, x)


def _mid_body(s0_ref, s1_ref, xs_ref, dinv_ref, w1_ref, b1_ref, w2_ref, gs_ref):
    agg = (s0_ref[...] + s1_ref[...] + xs_ref[...]) * dinv_ref[...]
    h = agg @ w1_ref[...] + b1_ref[...]
    h = jnp.maximum(h, 0.0)
    nrm = jnp.sqrt(jnp.sum(h * h, axis=1, keepdims=True))
    h = h / jnp.maximum(nrm, 1e-12)
    gs_ref[...] = (h @ w2_ref[...]) * dinv_ref[...]


def _mid_call(s0, s1, xs, dinv, w1, b1, w2, npad):
    n = s0.shape[0]
    d = xs.shape[1]
    dh = w1.shape[1]
    do = w2.shape[1]
    grid = n // _BR
    return pl.pallas_call(
        _mid_body,
        grid=(grid,),
        in_specs=[
            pl.BlockSpec((_BR, d), lambda i: (i, 0)),
            pl.BlockSpec((_BR, d), lambda i: (i, 0)),
            pl.BlockSpec((_BR, d), lambda i: (i, 0)),
            pl.BlockSpec((_BR, 1), lambda i: (i, 0)),
            pl.BlockSpec((d, dh), lambda i: (0, 0)),
            pl.BlockSpec((1, dh), lambda i: (0, 0)),
            pl.BlockSpec((dh, do), lambda i: (0, 0)),
        ],
        out_specs=pl.BlockSpec((_BR, do), lambda i: (i, 0)),
        out_shape=jax.ShapeDtypeStruct((npad, do), jnp.float32),
    )(s0, s1, xs, dinv, w1, b1, w2)


def _final_body(t0_ref, t1_ref, gs_ref, dinv_ref, b2_ref, out_ref):
    out_ref[...] = (t0_ref[...] + t1_ref[...] + gs_ref[...]) * dinv_ref[...] + b2_ref[...]


def _final_call(t0, t1, gs, dinv, b2):
    n, d = t0.shape
    grid = n // _BR
    return pl.pallas_call(
        _final_body,
        grid=(grid,),
        in_specs=[
            pl.BlockSpec((_BR, d), lambda i: (i, 0)),
            pl.BlockSpec((_BR, d), lambda i: (i, 0)),
            pl.BlockSpec((_BR, d), lambda i: (i, 0)),
            pl.BlockSpec((_BR, 1), lambda i: (i, 0)),
            pl.BlockSpec((1, d), lambda i: (0, 0)),
        ],
        out_specs=pl.BlockSpec((_BR, d), lambda i: (i, 0)),
        out_shape=jax.ShapeDtypeStruct((n, d), jnp.float32),
    )(t0, t1, gs, dinv, b2)


def kernel(x, edge_index, W1, b1, W2, b2):
    n, d_in = x.shape
    e = edge_index.shape[1]
    assert e % (NW * CH) == 0 and (e // CH // NW) % 2 == 0 and n % NS == 0

    src = edge_index[0]
    dst = edge_index[1]
    src2 = src.reshape(e // CH, CH)
    dst2 = dst.reshape(e // CH, CH)

    npad = ((n + NS * 16 - 1) // (NS * 16)) * (NS * 16)
    zeros_hist = jnp.zeros((16 * (npad // 2),), jnp.float32)
    zeros_feat = jnp.zeros((npad, d_in), jnp.float32)

    degp = _deg_kernel(dst, zeros_hist, npad)
    degp2 = degp.reshape(NC * npad, 1)
    d0 = lax.slice(degp2, (0, 0), (n, 1))
    d1 = lax.slice(degp2, (npad, 0), (npad + n, 1))

    dinv, xs = _scale_call(d0, d1, x, npad)

    sp = _spmm_kernel(xs, src2, dst2, zeros_feat)
    s0 = lax.slice(sp, (0, 0), (n, d_in))
    s1 = lax.slice(sp, (npad, 0), (npad + n, d_in))

    gs = _mid_call(s0, s1, xs, dinv, W1, b1.reshape(1, -1), W2, npad)

    tp = _spmm_kernel(gs, src2, dst2, zeros_feat)
    d_out = gs.shape[1]
    t0 = lax.slice(tp, (0, 0), (n, d_out))
    t1 = lax.slice(tp, (npad, 0), (npad + n, d_out))

    return _final_call(t0, t1, gs, dinv, b2.reshape(1, -1))


# speedup vs baseline: 27.1582x; 1.0025x over previous
import functools

import jax
import jax.numpy as jnp
from jax import lax
from jax.experimental import pallas as pl
from jax.experimental.pallas import tpu as pltpu
from jax.experimental.pallas import tpu_sc as plsc

NC = 2
NS = 16
NW = NC * NS

CHUNK = 80


def _mesh():
    return plsc.VectorSubcoreMesh(
        core_axis_name="c", subcore_axis_name="s", num_cores=NC, num_subcores=NS
    )


def _deg_kernel(dst, zeros_hist, npad):
    e = dst.shape[0]
    ept = e // NW
    nchunk = ept // CHUNK
    rpt = npad // NS
    half = npad // 2
    nlane = 16

    unroll = 5

    def body(dst_hbm, zh_hbm, out_hbm, dstv, hist, degv, tmp, outv, slots):
        c = lax.axis_index("c")
        s = lax.axis_index("s")
        wid = s * NC + c
        ones16 = jnp.full((nlane,), 1.0, jnp.float32)
        lane_ids = lax.iota(jnp.int32, nlane)
        lane_off = (lane_ids % 8) * npad
        m_lo = lane_ids < 8
        m_hi = lane_ids >= 8

        pltpu.sync_copy(dst_hbm.at[pl.ds(wid * ept, ept)], dstv)
        pltpu.sync_copy(zh_hbm, hist)

        def chunk(i, carry):
            for u in range(unroll):
                dv = dstv[pl.ds((i * unroll + u) * nlane, nlane)]
                idx = dv + lane_off
                plsc.addupdate_scatter(hist, [idx], ones16, mask=m_lo)
                plsc.addupdate_scatter(hist, [idx], ones16, mask=m_hi)
            return carry

        lax.fori_loop(0, ept // (nlane * unroll), chunk, 0)

        def reduce_blk(mb, carry):
            for u in range(2):
                acc = jnp.zeros((nlane,), jnp.float32)
                for l in range(8):
                    acc = acc + hist[pl.ds(l * npad + (2 * mb + u) * nlane, nlane)]
                degv[pl.ds((2 * mb + u) * nlane, nlane)] = acc
            return carry

        lax.fori_loop(0, npad // nlane // 2, reduce_blk, 0)

        pltpu.sync_copy(degv, slots.at[pl.ds(s * npad, npad)])
        plsc.subcore_barrier()
        for t in range(NS):
            pltpu.sync_copy(
                slots.at[pl.ds(t * npad + s * rpt, rpt)],
                tmp.at[pl.ds(t * rpt, rpt)],
            )

        def reduce_tiles(mb, carry):
            acc = jnp.zeros((nlane,), jnp.float32)
            for t in range(NS):
                acc = acc + tmp[pl.ds(t * rpt + mb * nlane, nlane)]
            outv[pl.ds(mb * nlane, nlane)] = acc
            return carry

        lax.fori_loop(0, rpt // nlane, reduce_tiles, 0)
        pltpu.sync_copy(outv, out_hbm.at[pl.ds(c * npad + s * rpt, rpt)])

    f = pl.kernel(
        body,
        out_type=jax.ShapeDtypeStruct((NC * npad,), jnp.float32),
        mesh=_mesh(),
        compiler_params=pltpu.CompilerParams(needs_layout_passes=False),
        scratch_types=[
            pltpu.VMEM((ept,), jnp.int32),
            pltpu.VMEM((nlane * half,), jnp.float32),
            pltpu.VMEM((npad,), jnp.float32),
            pltpu.VMEM((NS * rpt,), jnp.float32),
            pltpu.VMEM((rpt,), jnp.float32),
            pltpu.VMEM_SHARED((NS * npad,), jnp.float32),
        ],
    )
    return f(dst, zeros_hist)


CH = 125


def _spmm_kernel(table, src2, dst2, zeros_feat):
    n, d = table.shape
    nrows = src2.shape[0]
    cpt = nrows // NW
    rpt = n // NS

    def body(tab_hbm, src_hbm, dst_hbm, zeros_hbm, out_hbm,
             sidx, didx, rows0, rows1, gsem0, gsem1, ssem0, ssem1, acc):
        c = lax.axis_index("c")
        s = lax.axis_index("s")
        wid = s * NC + c
        pltpu.sync_copy(
            zeros_hbm.at[pl.ds(s * rpt, rpt)], acc.at[pl.ds(s * rpt, rpt)]
        )
        plsc.subcore_barrier()

        rows = (rows0, rows1)
        gsems = (gsem0, gsem1)
        ssems = (ssem0, ssem1)
        hcpt = cpt // 2

        def wait_gather(j, b):
            pltpu.make_async_copy(tab_hbm.at[sidx.at[j]], rows[b], gsems[b]).wait()

        def wait_scatter(j, b):
            pltpu.make_async_copy(rows[b], acc.at[didx.at[j]], ssems[b]).wait()

        for h in range(2):
            pltpu.sync_copy(src_hbm.at[pl.ds(wid * cpt + h * hcpt, hcpt)], sidx)
            pltpu.sync_copy(dst_hbm.at[pl.ds(wid * cpt + h * hcpt, hcpt)], didx)
            pltpu.async_copy(tab_hbm.at[sidx.at[0]], rows0, gsem0)
            wait_gather(0, 0)
            pltpu.async_copy(tab_hbm.at[sidx.at[1]], rows1, gsem1)
            pltpu.async_copy(rows0, acc.at[didx.at[0]], ssem0, add=True)

            def group(g, carry):
                for u in range(2):
                    j = 2 * g + 1 + u
                    b = 1 - u
                    nb = u
                    wait_gather(j, b)
                    wait_scatter(j - 1, nb)
                    pltpu.async_copy(tab_hbm.at[sidx.at[j + 1]], rows[nb], gsems[nb])
                    pltpu.async_copy(rows[b], acc.at[didx.at[j]], ssems[b], add=True)
                return carry

            lax.fori_loop(0, (hcpt - 2) // 2, group, 0)
            wait_gather(hcpt - 1, 1)
            wait_scatter(hcpt - 2, 0)
            pltpu.sync_copy(rows1, acc.at[didx.at[hcpt - 1]], add=True)
        plsc.subcore_barrier()
        pltpu.sync_copy(
            acc.at[pl.ds(s * rpt, rpt)],
            out_hbm.at[pl.ds(c * n + s * rpt, rpt)],
        )

    f = pl.kernel(
        body,
        out_type=jax.ShapeDtypeStruct((NC * n, d), jnp.float32),
        mesh=_mesh(),
        scratch_types=[
            pltpu.VMEM((cpt // 2, CH), jnp.int32),
            pltpu.VMEM((cpt // 2, CH), jnp.int32),
            pltpu.VMEM((CH, d), jnp.float32),
            pltpu.VMEM((CH, d), jnp.float32),
            pltpu.SemaphoreType.DMA,
            pltpu.SemaphoreType.DMA,
            pltpu.SemaphoreType.DMA,
            pltpu.SemaphoreType.DMA,
            pltpu.VMEM_SHARED((n, d), jnp.float32),
        ],
    )
    return f(table, src2, dst2, zeros_feat)


_BR = 640
_NB = 16


def _scale_body(d0_ref, d1_ref, x_ref, dinv_ref, xs_ref):
    deg = d0_ref[...] + d1_ref[...] + 1.0
    dv = lax.rsqrt(jnp.maximum(deg, 1e-12))
    dinv_ref[...] = dv
    xs_ref[...] = x_ref[...] * dv


def _scale_call(degp, x, npad):
    d = x.shape[1]
    return pl.pallas_call(
        _scale_body,
        grid=(_NB,),
        in_specs=[
            pl.BlockSpec((_BR, 1), lambda i: (i, 0)),
            pl.BlockSpec((_BR, 1), lambda i: (i + _NB, 0)),
            pl.BlockSpec((_BR, d), lambda i: (i, 0)),
        ],
        out_specs=[
            pl.BlockSpec((_BR, 1), lambda i: (i, 0)),
            pl.BlockSpec((_BR, d), lambda i: (i, 0)),
        ],
        out_shape=[
            jax.ShapeDtypeStruct((npad, 1), jnp.float32),
            jax.ShapeDtypeStruct((npad, d), jnp.float32),
        ],
    )(degp, degp, x)


def _mid_body(s0_ref, s1_ref, xs_ref, dinv_ref, w1_ref, b1_ref, w2_ref, gs_ref):
    agg = (s0_ref[...] + s1_ref[...] + xs_ref[...]) * dinv_ref[...]
    h = agg @ w1_ref[...] + b1_ref[...]
    h = jnp.maximum(h, 0.0)
    nrm = jnp.sqrt(jnp.sum(h * h, axis=1, keepdims=True))
    h = h / jnp.maximum(nrm, 1e-12)
    gs_ref[...] = (h @ w2_ref[...]) * dinv_ref[...]


def _mid_call(sp, xs, dinv, w1, b1, w2, npad):
    d = xs.shape[1]
    dh = w1.shape[1]
    do = w2.shape[1]
    return pl.pallas_call(
        _mid_body,
        grid=(_NB,),
        in_specs=[
            pl.BlockSpec((_BR, d), lambda i: (i, 0)),
            pl.BlockSpec((_BR, d), lambda i: (i + _NB, 0)),
            pl.BlockSpec((_BR, d), lambda i: (i, 0)),
            pl.BlockSpec((_BR, 1), lambda i: (i, 0)),
            pl.BlockSpec((d, dh), lambda i: (0, 0)),
            pl.BlockSpec((1, dh), lambda i: (0, 0)),
            pl.BlockSpec((dh, do), lambda i: (0, 0)),
        ],
        out_specs=pl.BlockSpec((_BR, do), lambda i: (i, 0)),
        out_shape=jax.ShapeDtypeStruct((npad, do), jnp.float32),
    )(sp, sp, xs, dinv, w1, b1, w2)


def _final_body(t0_ref, t1_ref, gs_ref, dinv_ref, b2_ref, out_ref):
    out_ref[...] = (t0_ref[...] + t1_ref[...] + gs_ref[...]) * dinv_ref[...] + b2_ref[...]


def _final_call(tp, gs, dinv, b2, npad):
    d = gs.shape[1]
    return pl.pallas_call(
        _final_body,
        grid=(_NB,),
        in_specs=[
            pl.BlockSpec((_BR, d), lambda i: (i, 0)),
            pl.BlockSpec((_BR, d), lambda i: (i + _NB, 0)),
            pl.BlockSpec((_BR, d), lambda i: (i, 0)),
            pl.BlockSpec((_BR, 1), lambda i: (i, 0)),
            pl.BlockSpec((1, d), lambda i: (0, 0)),
        ],
        out_specs=pl.BlockSpec((_BR, d), lambda i: (i, 0)),
        out_shape=jax.ShapeDtypeStruct((npad, d), jnp.float32),
    )(tp, tp, gs, dinv, b2)


def kernel(x, edge_index, W1, b1, W2, b2):
    n, d_in = x.shape
    e = edge_index.shape[1]
    assert e % (NW * CH) == 0 and (e // CH // NW) % 2 == 0 and n % NS == 0

    src = edge_index[0]
    dst = edge_index[1]
    src2 = src.reshape(e // CH, CH)
    dst2 = dst.reshape(e // CH, CH)

    npad = ((n + NS * 16 - 1) // (NS * 16)) * (NS * 16)
    zeros_hist = jnp.zeros((16 * (npad // 2),), jnp.float32)
    zeros_feat = jnp.zeros((npad, d_in), jnp.float32)

    degp = _deg_kernel(dst, zeros_hist, npad)

    dinv, xs = _scale_call(degp.reshape(NC * npad, 1), x, npad)

    sp = _spmm_kernel(xs, src2, dst2, zeros_feat)

    gs = _mid_call(sp, xs, dinv, W1, b1.reshape(1, -1), W2, npad)

    tp = _spmm_kernel(gs, src2, dst2, zeros_feat)

    out = _final_call(tp, gs, dinv, b2.reshape(1, -1), npad)
    return lax.slice(out, (0, 0), (n, gs.shape[1]))


# speedup vs baseline: 27.3045x; 1.0054x over previous
import functools

import jax
import jax.numpy as jnp
from jax import lax
from jax.experimental import pallas as pl
from jax.experimental.pallas import tpu as pltpu
from jax.experimental.pallas import tpu_sc as plsc

NC = 2
NS = 16
NW = NC * NS

CHUNK = 80


def _mesh():
    return plsc.VectorSubcoreMesh(
        core_axis_name="c", subcore_axis_name="s", num_cores=NC, num_subcores=NS
    )


def _deg_kernel(dst, zeros_hist, npad):
    e = dst.shape[0]
    ept = e // NW
    nchunk = ept // CHUNK
    rpt = npad // NS
    half = npad // 2
    nlane = 16

    unroll = 5

    def body(dst_hbm, zh_hbm, out_hbm, dstv, hist, degv, tmp, outv, slots):
        c = lax.axis_index("c")
        s = lax.axis_index("s")
        wid = s * NC + c
        ones16 = jnp.full((nlane,), 1.0, jnp.float32)
        lane_ids = lax.iota(jnp.int32, nlane)
        lane_off = (lane_ids % 8) * npad
        m_lo = lane_ids < 8
        m_hi = lane_ids >= 8

        pltpu.sync_copy(dst_hbm.at[pl.ds(wid * ept, ept)], dstv)
        pltpu.sync_copy(zh_hbm, hist)

        def chunk(i, carry):
            for u in range(unroll):
                dv = dstv[pl.ds((i * unroll + u) * nlane, nlane)]
                idx = dv + lane_off
                plsc.addupdate_scatter(hist, [idx], ones16, mask=m_lo)
                plsc.addupdate_scatter(hist, [idx], ones16, mask=m_hi)
            return carry

        lax.fori_loop(0, ept // (nlane * unroll), chunk, 0)

        def reduce_blk(mb, carry):
            for u in range(2):
                acc = jnp.zeros((nlane,), jnp.float32)
                for l in range(8):
                    acc = acc + hist[pl.ds(l * npad + (2 * mb + u) * nlane, nlane)]
                degv[pl.ds((2 * mb + u) * nlane, nlane)] = acc
            return carry

        lax.fori_loop(0, npad // nlane // 2, reduce_blk, 0)

        pltpu.sync_copy(degv, slots.at[pl.ds(s * npad, npad)])
        plsc.subcore_barrier()
        for t in range(NS):
            pltpu.sync_copy(
                slots.at[pl.ds(t * npad + s * rpt, rpt)],
                tmp.at[pl.ds(t * rpt, rpt)],
            )

        def reduce_tiles(mb, carry):
            acc = jnp.zeros((nlane,), jnp.float32)
            for t in range(NS):
                acc = acc + tmp[pl.ds(t * rpt + mb * nlane, nlane)]
            outv[pl.ds(mb * nlane, nlane)] = acc
            return carry

        lax.fori_loop(0, rpt // nlane, reduce_tiles, 0)
        pltpu.sync_copy(outv, out_hbm.at[pl.ds(c * npad + s * rpt, rpt)])

    f = pl.kernel(
        body,
        out_type=jax.ShapeDtypeStruct((NC * npad,), jnp.float32),
        mesh=_mesh(),
        compiler_params=pltpu.CompilerParams(needs_layout_passes=False),
        scratch_types=[
            pltpu.VMEM((ept,), jnp.int32),
            pltpu.VMEM((nlane * half,), jnp.float32),
            pltpu.VMEM((npad,), jnp.float32),
            pltpu.VMEM((NS * rpt,), jnp.float32),
            pltpu.VMEM((rpt,), jnp.float32),
            pltpu.VMEM_SHARED((NS * npad,), jnp.float32),
        ],
    )
    return f(dst, zeros_hist)


CH = 125


def _spmm_kernel(table, src2, dst2, zeros_feat):
    n, d = table.shape
    nrows = src2.shape[0]
    cpt = nrows // NW
    rpt = n // NS

    def body(tab_hbm, src_hbm, dst_hbm, zeros_hbm, out_hbm,
             sidx, didx, rows0, rows1, gsem0, gsem1, ssem0, ssem1, acc):
        c = lax.axis_index("c")
        s = lax.axis_index("s")
        wid = s * NC + c
        @pl.when(c == 0)
        def _():
            pltpu.sync_copy(
                tab_hbm.at[pl.ds(s * rpt, rpt)], acc.at[pl.ds(s * rpt, rpt)]
            )
        @pl.when(c != 0)
        def _():
            pltpu.sync_copy(
                zeros_hbm.at[pl.ds(s * rpt, rpt)], acc.at[pl.ds(s * rpt, rpt)]
            )
        plsc.subcore_barrier()

        rows = (rows0, rows1)
        gsems = (gsem0, gsem1)
        ssems = (ssem0, ssem1)
        hcpt = cpt // 2

        def wait_gather(j, b):
            pltpu.make_async_copy(tab_hbm.at[sidx.at[j]], rows[b], gsems[b]).wait()

        def wait_scatter(j, b):
            pltpu.make_async_copy(rows[b], acc.at[didx.at[j]], ssems[b]).wait()

        for h in range(2):
            pltpu.sync_copy(src_hbm.at[pl.ds(wid * cpt + h * hcpt, hcpt)], sidx)
            pltpu.sync_copy(dst_hbm.at[pl.ds(wid * cpt + h * hcpt, hcpt)], didx)
            pltpu.async_copy(tab_hbm.at[sidx.at[0]], rows0, gsem0)
            wait_gather(0, 0)
            pltpu.async_copy(tab_hbm.at[sidx.at[1]], rows1, gsem1)
            pltpu.async_copy(rows0, acc.at[didx.at[0]], ssem0, add=True)

            def group(g, carry):
                for u in range(2):
                    j = 2 * g + 1 + u
                    b = 1 - u
                    nb = u
                    wait_gather(j, b)
                    wait_scatter(j - 1, nb)
                    pltpu.async_copy(tab_hbm.at[sidx.at[j + 1]], rows[nb], gsems[nb])
                    pltpu.async_copy(rows[b], acc.at[didx.at[j]], ssems[b], add=True)
                return carry

            lax.fori_loop(0, (hcpt - 2) // 2, group, 0)
            wait_gather(hcpt - 1, 1)
            wait_scatter(hcpt - 2, 0)
            pltpu.sync_copy(rows1, acc.at[didx.at[hcpt - 1]], add=True)
        plsc.subcore_barrier()
        pltpu.sync_copy(
            acc.at[pl.ds(s * rpt, rpt)],
            out_hbm.at[pl.ds(c * n + s * rpt, rpt)],
        )

    f = pl.kernel(
        body,
        out_type=jax.ShapeDtypeStruct((NC * n, d), jnp.float32),
        mesh=_mesh(),
        scratch_types=[
            pltpu.VMEM((cpt // 2, CH), jnp.int32),
            pltpu.VMEM((cpt // 2, CH), jnp.int32),
            pltpu.VMEM((CH, d), jnp.float32),
            pltpu.VMEM((CH, d), jnp.float32),
            pltpu.SemaphoreType.DMA,
            pltpu.SemaphoreType.DMA,
            pltpu.SemaphoreType.DMA,
            pltpu.SemaphoreType.DMA,
            pltpu.VMEM_SHARED((n, d), jnp.float32),
        ],
    )
    return f(table, src2, dst2, zeros_feat)


_BR = 640
_NB = 16


def _scale_body(d0_ref, d1_ref, x_ref, dinv_ref, xs_ref):
    deg = d0_ref[...] + d1_ref[...] + 1.0
    dv = lax.rsqrt(jnp.maximum(deg, 1e-12))
    dinv_ref[...] = dv
    xs_ref[...] = x_ref[...] * dv


def _scale_call(degp, x, npad):
    d = x.shape[1]
    return pl.pallas_call(
        _scale_body,
        grid=(_NB,),
        in_specs=[
            pl.BlockSpec((_BR, 1), lambda i: (i, 0)),
            pl.BlockSpec((_BR, 1), lambda i: (i + _NB, 0)),
            pl.BlockSpec((_BR, d), lambda i: (i, 0)),
        ],
        out_specs=[
            pl.BlockSpec((_BR, 1), lambda i: (i, 0)),
            pl.BlockSpec((_BR, d), lambda i: (i, 0)),
        ],
        out_shape=[
            jax.ShapeDtypeStruct((npad, 1), jnp.float32),
            jax.ShapeDtypeStruct((npad, d), jnp.float32),
        ],
    )(degp, degp, x)


def _mid_body(s0_ref, s1_ref, dinv_ref, w1_ref, b1_ref, w2_ref, gs_ref):
    agg = (s0_ref[...] + s1_ref[...]) * dinv_ref[...]
    h = agg @ w1_ref[...] + b1_ref[...]
    h = jnp.maximum(h, 0.0)
    nrm = jnp.sqrt(jnp.sum(h * h, axis=1, keepdims=True))
    h = h / jnp.maximum(nrm, 1e-12)
    gs_ref[...] = (h @ w2_ref[...]) * dinv_ref[...]


def _mid_call(sp, dinv, w1, b1, w2, npad):
    d = sp.shape[1]
    dh = w1.shape[1]
    do = w2.shape[1]
    return pl.pallas_call(
        _mid_body,
        grid=(_NB,),
        in_specs=[
            pl.BlockSpec((_BR, d), lambda i: (i, 0)),
            pl.BlockSpec((_BR, d), lambda i: (i + _NB, 0)),
            pl.BlockSpec((_BR, 1), lambda i: (i, 0)),
            pl.BlockSpec((d, dh), lambda i: (0, 0)),
            pl.BlockSpec((1, dh), lambda i: (0, 0)),
            pl.BlockSpec((dh, do), lambda i: (0, 0)),
        ],
        out_specs=pl.BlockSpec((_BR, do), lambda i: (i, 0)),
        out_shape=jax.ShapeDtypeStruct((npad, do), jnp.float32),
    )(sp, sp, dinv, w1, b1, w2)


def _final_body(t0_ref, t1_ref, dinv_ref, b2_ref, out_ref):
    out_ref[...] = (t0_ref[...] + t1_ref[...]) * dinv_ref[...] + b2_ref[...]


def _final_call(tp, dinv, b2, npad, d):
    return pl.pallas_call(
        _final_body,
        grid=(_NB,),
        in_specs=[
            pl.BlockSpec((_BR, d), lambda i: (i, 0)),
            pl.BlockSpec((_BR, d), lambda i: (i + _NB, 0)),
            pl.BlockSpec((_BR, 1), lambda i: (i, 0)),
            pl.BlockSpec((1, d), lambda i: (0, 0)),
        ],
        out_specs=pl.BlockSpec((_BR, d), lambda i: (i, 0)),
        out_shape=jax.ShapeDtypeStruct((npad, d), jnp.float32),
    )(tp, tp, dinv, b2)


def kernel(x, edge_index, W1, b1, W2, b2):
    n, d_in = x.shape
    e = edge_index.shape[1]
    assert e % (NW * CH) == 0 and (e // CH // NW) % 2 == 0 and n % NS == 0

    src = edge_index[0]
    dst = edge_index[1]
    src2 = src.reshape(e // CH, CH)
    dst2 = dst.reshape(e // CH, CH)

    npad = ((n + NS * 16 - 1) // (NS * 16)) * (NS * 16)
    zeros_hist = jnp.zeros((16 * (npad // 2),), jnp.float32)
    zeros_feat = jnp.zeros((npad, d_in), jnp.float32)

    degp = _deg_kernel(dst, zeros_hist, npad)

    dinv, xs = _scale_call(degp.reshape(NC * npad, 1), x, npad)

    sp = _spmm_kernel(xs, src2, dst2, zeros_feat)

    gs = _mid_call(sp, dinv, W1, b1.reshape(1, -1), W2, npad)

    tp = _spmm_kernel(gs, src2, dst2, zeros_feat)

    out = _final_call(tp, dinv, b2.reshape(1, -1), npad, gs.shape[1])
    return lax.slice(out, (0, 0), (n, gs.shape[1]))


# speedup vs baseline: 27.3093x; 1.0002x over previous
import jax
import jax.numpy as jnp
from jax import lax
from jax.experimental import pallas as pl
from jax.experimental.pallas import tpu as pltpu
from jax.experimental.pallas import tpu_sc as plsc

NC = 2
NS = 16
NW = NC * NS


def _mesh():
    return plsc.VectorSubcoreMesh(
        core_axis_name="c", subcore_axis_name="s", num_cores=NC, num_subcores=NS
    )


def _deg_kernel(dst, zeros_hist, npad):
    e = dst.shape[0]
    ept = e // NW
    rpt = npad // NS
    half = npad // 2
    nlane = 16

    unroll = 5

    def body(dst_hbm, zh_hbm, out_hbm, dstv, hist, degv, tmp, outv, slots):
        c = lax.axis_index("c")
        s = lax.axis_index("s")
        wid = s * NC + c
        ones16 = jnp.full((nlane,), 1.0, jnp.float32)
        lane_ids = lax.iota(jnp.int32, nlane)
        lane_off = (lane_ids % 8) * npad
        m_lo = lane_ids < 8
        m_hi = lane_ids >= 8

        pltpu.sync_copy(dst_hbm.at[pl.ds(wid * ept, ept)], dstv)
        pltpu.sync_copy(zh_hbm, hist)

        def chunk(i, carry):
            for u in range(unroll):
                dv = dstv[pl.ds((i * unroll + u) * nlane, nlane)]
                idx = dv + lane_off
                plsc.addupdate_scatter(hist, [idx], ones16, mask=m_lo)
                plsc.addupdate_scatter(hist, [idx], ones16, mask=m_hi)
            return carry

        lax.fori_loop(0, ept // (nlane * unroll), chunk, 0)

        def reduce_blk(mb, carry):
            for u in range(2):
                acc = jnp.zeros((nlane,), jnp.float32)
                for l in range(8):
                    acc = acc + hist[pl.ds(l * npad + (2 * mb + u) * nlane, nlane)]
                degv[pl.ds((2 * mb + u) * nlane, nlane)] = acc
            return carry

        lax.fori_loop(0, npad // nlane // 2, reduce_blk, 0)

        pltpu.sync_copy(degv, slots.at[pl.ds(s * npad, npad)])
        plsc.subcore_barrier()
        for t in range(NS):
            pltpu.sync_copy(
                slots.at[pl.ds(t * npad + s * rpt, rpt)],
                tmp.at[pl.ds(t * rpt, rpt)],
            )

        def reduce_tiles(mb, carry):
            acc = jnp.zeros((nlane,), jnp.float32)
            for t in range(NS):
                acc = acc + tmp[pl.ds(t * rpt + mb * nlane, nlane)]
            outv[pl.ds(mb * nlane, nlane)] = acc
            return carry

        lax.fori_loop(0, rpt // nlane, reduce_tiles, 0)
        pltpu.sync_copy(outv, out_hbm.at[pl.ds(c * npad + s * rpt, rpt)])

    f = pl.kernel(
        body,
        out_type=jax.ShapeDtypeStruct((NC * npad,), jnp.float32),
        mesh=_mesh(),
        compiler_params=pltpu.CompilerParams(needs_layout_passes=False),
        scratch_types=[
            pltpu.VMEM((ept,), jnp.int32),
            pltpu.VMEM((nlane * half,), jnp.float32),
            pltpu.VMEM((npad,), jnp.float32),
            pltpu.VMEM((NS * rpt,), jnp.float32),
            pltpu.VMEM((rpt,), jnp.float32),
            pltpu.VMEM_SHARED((NS * npad,), jnp.float32),
        ],
    )
    return f(dst, zeros_hist)


CH = 125


def _spmm_kernel(table, src2, dst2, zeros_feat):
    n, d = table.shape
    nrows = src2.shape[0]
    cpt = nrows // NW
    rpt = n // NS

    def body(tab_hbm, src_hbm, dst_hbm, zeros_hbm, out_hbm,
             sidx, didx, rows0, rows1, gsem0, gsem1, ssem0, ssem1, acc):
        c = lax.axis_index("c")
        s = lax.axis_index("s")
        wid = s * NC + c
        @pl.when(c == 0)
        def _():
            pltpu.sync_copy(
                tab_hbm.at[pl.ds(s * rpt, rpt)], acc.at[pl.ds(s * rpt, rpt)]
            )
        @pl.when(c != 0)
        def _():
            pltpu.sync_copy(
                zeros_hbm.at[pl.ds(s * rpt, rpt)], acc.at[pl.ds(s * rpt, rpt)]
            )
        plsc.subcore_barrier()

        rows = (rows0, rows1)
        gsems = (gsem0, gsem1)
        ssems = (ssem0, ssem1)
        hcpt = cpt // 2

        def wait_gather(j, b):
            pltpu.make_async_copy(tab_hbm.at[sidx.at[j]], rows[b], gsems[b]).wait()

        def wait_scatter(j, b):
            pltpu.make_async_copy(rows[b], acc.at[didx.at[j]], ssems[b]).wait()

        for h in range(2):
            pltpu.sync_copy(src_hbm.at[pl.ds(wid * cpt + h * hcpt, hcpt)], sidx)
            pltpu.sync_copy(dst_hbm.at[pl.ds(wid * cpt + h * hcpt, hcpt)], didx)
            pltpu.async_copy(tab_hbm.at[sidx.at[0]], rows0, gsem0)
            wait_gather(0, 0)
            pltpu.async_copy(tab_hbm.at[sidx.at[1]], rows1, gsem1)
            pltpu.async_copy(rows0, acc.at[didx.at[0]], ssem0, add=True)

            def group(g, carry):
                for u in range(2):
                    j = 2 * g + 1 + u
                    b = 1 - u
                    nb = u
                    wait_gather(j, b)
                    wait_scatter(j - 1, nb)
                    pltpu.async_copy(tab_hbm.at[sidx.at[j + 1]], rows[nb], gsems[nb])
                    pltpu.async_copy(rows[b], acc.at[didx.at[j]], ssems[b], add=True)
                return carry

            lax.fori_loop(0, (hcpt - 2) // 2, group, 0)
            wait_gather(hcpt - 1, 1)
            wait_scatter(hcpt - 2, 0)
            pltpu.sync_copy(rows1, acc.at[didx.at[hcpt - 1]], add=True)
        plsc.subcore_barrier()
        pltpu.sync_copy(
            acc.at[pl.ds(s * rpt, rpt)],
            out_hbm.at[pl.ds(c * n + s * rpt, rpt)],
        )

    f = pl.kernel(
        body,
        out_type=jax.ShapeDtypeStruct((NC * n, d), jnp.float32),
        mesh=_mesh(),
        scratch_types=[
            pltpu.VMEM((cpt // 2, CH), jnp.int32),
            pltpu.VMEM((cpt // 2, CH), jnp.int32),
            pltpu.VMEM((CH, d), jnp.float32),
            pltpu.VMEM((CH, d), jnp.float32),
            pltpu.SemaphoreType.DMA,
            pltpu.SemaphoreType.DMA,
            pltpu.SemaphoreType.DMA,
            pltpu.SemaphoreType.DMA,
            pltpu.VMEM_SHARED((n, d), jnp.float32),
        ],
    )
    return f(table, src2, dst2, zeros_feat)


_BR = 640
_NB = 16


def _scale_body(d0_ref, d1_ref, x_ref, dinv_ref, xs_ref):
    deg = d0_ref[...] + d1_ref[...] + 1.0
    dv = lax.rsqrt(jnp.maximum(deg, 1e-12))
    dinv_ref[...] = dv
    xs_ref[...] = x_ref[...] * dv


def _scale_call(degp, x, npad):
    d = x.shape[1]
    return pl.pallas_call(
        _scale_body,
        grid=(_NB,),
        in_specs=[
            pl.BlockSpec((_BR, 1), lambda i: (i, 0)),
            pl.BlockSpec((_BR, 1), lambda i: (i + _NB, 0)),
            pl.BlockSpec((_BR, d), lambda i: (i, 0)),
        ],
        out_specs=[
            pl.BlockSpec((_BR, 1), lambda i: (i, 0)),
            pl.BlockSpec((_BR, d), lambda i: (i, 0)),
        ],
        out_shape=[
            jax.ShapeDtypeStruct((npad, 1), jnp.float32),
            jax.ShapeDtypeStruct((npad, d), jnp.float32),
        ],
    )(degp, degp, x)


def _mid_body(s0_ref, s1_ref, dinv_ref, w1_ref, b1_ref, w2_ref, gs_ref):
    agg = (s0_ref[...] + s1_ref[...]) * dinv_ref[...]
    h = agg @ w1_ref[...] + b1_ref[...]
    h = jnp.maximum(h, 0.0)
    nrm = jnp.sqrt(jnp.sum(h * h, axis=1, keepdims=True))
    h = h / jnp.maximum(nrm, 1e-12)
    gs_ref[...] = (h @ w2_ref[...]) * dinv_ref[...]


def _mid_call(sp, dinv, w1, b1, w2, npad):
    d = sp.shape[1]
    dh = w1.shape[1]
    do = w2.shape[1]
    return pl.pallas_call(
        _mid_body,
        grid=(_NB,),
        in_specs=[
            pl.BlockSpec((_BR, d), lambda i: (i, 0)),
            pl.BlockSpec((_BR, d), lambda i: (i + _NB, 0)),
            pl.BlockSpec((_BR, 1), lambda i: (i, 0)),
            pl.BlockSpec((d, dh), lambda i: (0, 0)),
            pl.BlockSpec((1, dh), lambda i: (0, 0)),
            pl.BlockSpec((dh, do), lambda i: (0, 0)),
        ],
        out_specs=pl.BlockSpec((_BR, do), lambda i: (i, 0)),
        out_shape=jax.ShapeDtypeStruct((npad, do), jnp.float32),
    )(sp, sp, dinv, w1, b1, w2)


def _final_body(t0_ref, t1_ref, dinv_ref, b2_ref, out_ref):
    out_ref[...] = (t0_ref[...] + t1_ref[...]) * dinv_ref[...] + b2_ref[...]


def _final_call(tp, dinv, b2, npad, d):
    return pl.pallas_call(
        _final_body,
        grid=(_NB,),
        in_specs=[
            pl.BlockSpec((_BR, d), lambda i: (i, 0)),
            pl.BlockSpec((_BR, d), lambda i: (i + _NB, 0)),
            pl.BlockSpec((_BR, 1), lambda i: (i, 0)),
            pl.BlockSpec((1, d), lambda i: (0, 0)),
        ],
        out_specs=pl.BlockSpec((_BR, d), lambda i: (i, 0)),
        out_shape=jax.ShapeDtypeStruct((npad, d), jnp.float32),
    )(tp, tp, dinv, b2)


def kernel(x, edge_index, W1, b1, W2, b2):
    n, d_in = x.shape
    e = edge_index.shape[1]
    assert e % (NW * CH) == 0 and (e // CH // NW) % 2 == 0 and n % NS == 0

    src = edge_index[0]
    dst = edge_index[1]
    src2 = src.reshape(e // CH, CH)
    dst2 = dst.reshape(e // CH, CH)

    npad = ((n + NS * 16 - 1) // (NS * 16)) * (NS * 16)
    zeros_hist = jnp.zeros((16 * (npad // 2),), jnp.float32)
    zeros_feat = jnp.zeros((npad, d_in), jnp.float32)

    degp = _deg_kernel(dst, zeros_hist, npad)

    dinv, xs = _scale_call(degp.reshape(NC * npad, 1), x, npad)

    sp = _spmm_kernel(xs, src2, dst2, zeros_feat)

    gs = _mid_call(sp, dinv, W1, b1.reshape(1, -1), W2, npad)

    tp = _spmm_kernel(gs, src2, dst2, zeros_feat)

    out = _final_call(tp, dinv, b2.reshape(1, -1), npad, gs.shape[1])
    return lax.slice(out, (0, 0), (n, gs.shape[1]))
